# Initial kernel scaffold; baseline (speedup 1.0000x reference)
import jax
import jax.numpy as jnp
from jax.experimental import pallas as pl


def kernel(table, seq_len):
    raise NotImplementedError("write your pallas kernel here")


# speedup vs baseline: 6.2529x; 6.2529x over previous
import functools

import jax
import jax.numpy as jnp
from jax import lax
from jax.experimental import pallas as pl
from jax.experimental.pallas import tpu as pltpu
from jax.experimental.pallas import tpu_sc as plsc

EMBED = 32
MAX_REL = 128
VOCAB = 2 * MAX_REL + 1
SEQ = 2048

_INFO = plsc.get_sparse_core_info()
NC = _INFO.num_cores
NS = _INFO.num_subcores
NW = NC * NS
ROWS_PER_W = SEQ // NW
WIN = SEQ + ROWS_PER_W - 1
WIN_PAD = 2176
N_CHUNK = WIN_PAD // 128

_mesh = plsc.VectorSubcoreMesh(core_axis_name="c", subcore_axis_name="s")


@functools.partial(
    pl.kernel,
    out_type=jax.ShapeDtypeStruct((SEQ, SEQ, EMBED), jnp.float32),
    mesh=_mesh,
    compiler_params=pltpu.CompilerParams(use_tc_tiling_on_sc=False),
    scratch_types=[
        pltpu.VMEM((N_CHUNK, 128), jnp.int32),
        pltpu.VMEM((WIN_PAD, EMBED), jnp.float32),
        pltpu.SemaphoreType.DMA,
        pltpu.SemaphoreType.DMA,
    ],
)
def _rpe_sc(table_hbm, out_hbm, idx_v, band_v, gsem, csem):
    wid = lax.axis_index("s") * NC + lax.axis_index("c")
    row0 = wid * ROWS_PER_W
    win0 = (SEQ - 1) - (row0 + ROWS_PER_W - 1)

    lane = lax.iota(jnp.int32, 16)
    lo = SEQ - 1 - MAX_REL
    hi = SEQ - 1 + MAX_REL
    for c in range(N_CHUNK):
        def _fill(k, _, c=c):
            m = win0 + c * 128 + k * 16 + lane
            idx_v[c, pl.ds(k * 16, 16)] = jnp.clip(m, lo, hi) - lo
            return 0
        lax.fori_loop(0, 8, _fill, 0)

    gathers = [
        pltpu.make_async_copy(
            table_hbm.at[idx_v.at[c]],
            band_v.at[pl.ds(c * 128, 128)],
            gsem,
        )
        for c in range(N_CHUNK)
    ]
    for g in gathers:
        g.start()
    for g in gathers:
        g.wait()

    def _copy(r):
        return pltpu.make_async_copy(
            band_v.at[pl.ds(ROWS_PER_W - 1 - r, SEQ)],
            out_hbm.at[row0 + r],
            csem,
        )

    _copy(0).start()

    def _step(r, _):
        _copy(r).start()
        _copy(r - 1).wait()
        return 0

    lax.fori_loop(1, ROWS_PER_W, _step, 0)
    _copy(ROWS_PER_W - 1).wait()


def kernel(table, seq_len):
    return _rpe_sc(table)


# speedup vs baseline: 50.1365x; 8.0181x over previous
import functools

import jax
import jax.numpy as jnp
from jax import lax
from jax.experimental import pallas as pl
from jax.experimental.pallas import tpu as pltpu
from jax.experimental.pallas import tpu_sc as plsc

EMBED = 32
MAX_REL = 128
VOCAB = 2 * MAX_REL + 1
SEQ = 2048
LO = SEQ - 1 - MAX_REL
HI = SEQ - 1 + MAX_REL

_INFO = plsc.get_sparse_core_info()
NC = _INFO.num_cores
NS = _INFO.num_subcores
NW = NC * NS
N_ET = EMBED // 8
N_JT = SEQ // 128
NTILES = 23

_mesh = plsc.VectorSubcoreMesh(core_axis_name="c", subcore_axis_name="s")


@functools.partial(
    pl.kernel,
    out_type=jax.ShapeDtypeStruct((SEQ, N_ET, N_JT, 8, 128), jnp.float32),
    mesh=_mesh,
    compiler_params=pltpu.CompilerParams(needs_layout_passes=False),
    scratch_types=[
        pltpu.VMEM((VOCAB * EMBED,), jnp.float32),
        pltpu.VMEM((NTILES, EMBED, 128), jnp.float32),
        pltpu.SemaphoreType.DMA,
        pltpu.SemaphoreType.DMA,
    ],
)
def _rpe_sc(table_hbm, out_hbm, tbl_v, btt_v, lsem, csem):
    wid = lax.axis_index("s") * NC + lax.axis_index("c")

    pltpu.make_async_copy(table_hbm, tbl_v, lsem).start()

    lane = lax.iota(jnp.int32, 16)

    def build_tiles(p_lo, p_hi, virt0, base):
        def _tile(p, _):
            def _row(e, __):
                for kk in range(8):
                    m = base + 128 * (p + virt0) + kk * 16 + lane
                    ridx = jnp.clip(m, LO, HI) - LO
                    btt_v[p, e, pl.ds(kk * 16, 16)] = plsc.load_gather(
                        tbl_v, [ridx * EMBED + e])
                return 0
            lax.fori_loop(0, EMBED, _row, 0)
            return 0
        lax.fori_loop(p_lo, p_hi, _tile, 0)

    def copy_desc(k, et, virt0, c):
        i = c + 128 * k
        return pltpu.make_async_copy(
            btt_v.at[pl.ds(15 - k - virt0, 16), pl.ds(et * 8, 8), :],
            out_hbm.at[i, et],
            csem,
        )

    first = True
    for p in range(4):
        for h in range(2):
            virt0 = 8 * (1 - h)
            c = 4 * wid + p
            base = 127 - c
            if first:
                pltpu.make_async_copy(table_hbm, tbl_v, lsem).wait()
                build_tiles(0, NTILES, virt0, base)
                first = False
            else:
                build_tiles(6, 17, virt0, base)

            k0 = 8 * h
            for et in range(N_ET):
                copy_desc(k0, et, virt0, c).start()

            def _step(k, _, virt0=virt0, c=c):
                for et in range(N_ET):
                    copy_desc(k, et, virt0, c).start()
                for et in range(N_ET):
                    copy_desc(k - 1, et, virt0, c).wait()
                return 0

            lax.fori_loop(k0 + 1, k0 + 8, _step, 0)
            for et in range(N_ET):
                copy_desc(k0 + 7, et, virt0, c).wait()


def kernel(table, seq_len):
    out5 = _rpe_sc(table.reshape(VOCAB * EMBED))
    return out5.transpose(0, 2, 4, 1, 3).reshape(SEQ, SEQ, EMBED)


# speedup vs baseline: 64.5333x; 1.2872x over previous
import functools

import jax
import jax.numpy as jnp
from jax import lax
from jax.experimental import pallas as pl
from jax.experimental.pallas import tpu as pltpu
from jax.experimental.pallas import tpu_sc as plsc

EMBED = 32
MAX_REL = 128
VOCAB = 2 * MAX_REL + 1
SEQ = 2048
LO = SEQ - 1 - MAX_REL
HI = SEQ - 1 + MAX_REL

_INFO = plsc.get_sparse_core_info()
NC = _INFO.num_cores
NS = _INFO.num_subcores
NW = NC * NS
N_ET = EMBED // 8
N_JT = SEQ // 128
NTILES = 23

_mesh = plsc.VectorSubcoreMesh(core_axis_name="c", subcore_axis_name="s")


@functools.partial(
    pl.kernel,
    out_type=jax.ShapeDtypeStruct((SEQ, N_ET, N_JT, 8, 128), jnp.float32),
    mesh=_mesh,
    compiler_params=pltpu.CompilerParams(needs_layout_passes=False),
    scratch_types=[
        pltpu.VMEM((VOCAB * EMBED,), jnp.float32),
        pltpu.VMEM((NTILES, EMBED, 128), jnp.float32),
        pltpu.SemaphoreType.DMA,
        pltpu.SemaphoreType.DMA,
    ],
)
def _rpe_sc(table_hbm, out_hbm, tbl_v, btt_v, lsem, csem):
    wid = lax.axis_index("s") * NC + lax.axis_index("c")

    pltpu.make_async_copy(table_hbm, tbl_v, lsem).start()

    lane = lax.iota(jnp.int32, 16)

    def gather_tiles(p_lo, p_hi, virt0, base):
        def _tile(p, _):
            def _row(e, __):
                for kk in range(8):
                    m = base + 128 * (p + virt0) + kk * 16 + lane
                    ridx = jnp.clip(m, LO, HI) - LO
                    btt_v[p, e, pl.ds(kk * 16, 16)] = plsc.load_gather(
                        tbl_v, [ridx * EMBED + e])
                return 0
            lax.fori_loop(0, EMBED, _row, 0)
            return 0
        lax.fori_loop(p_lo, p_hi, _tile, 0)

    def fill_tiles(p_lo, p_hi, row):
        def _row(e, _):
            v = plsc.load_gather(tbl_v, [jnp.full((16,), row * EMBED, jnp.int32) + e])
            for p in range(p_lo, p_hi):
                for kk in range(8):
                    btt_v[p, e, pl.ds(kk * 16, 16)] = v
            return 0
        lax.fori_loop(0, EMBED, _row, 0)

    def copy_desc(d, virt0, c, k0):
        k = k0 + lax.shift_right_logical(d, 2)
        et = lax.bitwise_and(d, 3)
        return pltpu.make_async_copy(
            btt_v.at[pl.ds(15 - k - virt0, 16), pl.ds(et * 8, 8), :],
            out_hbm.at[c + 128 * k, et],
            csem,
        )

    halves = [0, 1, 1, 0, 0, 1, 1, 0]
    prev_h = None
    for idx, h in enumerate(halves):
        p = idx // 2
        c = 4 * wid + p
        base = 127 - c
        virt0 = 8 * (1 - h)
        band_lo = 14 - virt0

        if prev_h is None:
            pltpu.make_async_copy(table_hbm, tbl_v, lsem).wait()
            fill_tiles(0, 6, 0)
            fill_tiles(9, NTILES, VOCAB - 1)
        elif h != prev_h:
            if h == 1:
                fill_tiles(6, 14, 0)
            else:
                fill_tiles(9, 17, VOCAB - 1)
        gather_tiles(band_lo, band_lo + 3, virt0, base)
        prev_h = h

        k0 = 8 * h

        def _start(d, _, virt0=virt0, c=c, k0=k0):
            copy_desc(d, virt0, c, k0).start()
            return 0

        def _roll(d, _, virt0=virt0, c=c, k0=k0):
            copy_desc(d, virt0, c, k0).start()
            copy_desc(d - 8, virt0, c, k0).wait()
            return 0

        def _drain(d, _, virt0=virt0, c=c, k0=k0):
            copy_desc(d, virt0, c, k0).wait()
            return 0

        lax.fori_loop(0, 8, _start, 0)
        lax.fori_loop(8, 32, _roll, 0)
        lax.fori_loop(24, 32, _drain, 0)


def kernel(table, seq_len):
    out5 = _rpe_sc(table.reshape(VOCAB * EMBED))
    return out5.transpose(0, 2, 4, 1, 3).reshape(SEQ, SEQ, EMBED)


# speedup vs baseline: 79.0167x; 1.2244x over previous
import functools

import jax
import jax.numpy as jnp
from jax import lax
from jax.experimental import pallas as pl
from jax.experimental.pallas import tpu as pltpu
from jax.experimental.pallas import tpu_sc as plsc

EMBED = 32
MAX_REL = 128
VOCAB = 2 * MAX_REL + 1
SEQ = 2048
LO = SEQ - 1 - MAX_REL
HI = SEQ - 1 + MAX_REL

_INFO = plsc.get_sparse_core_info()
NC = _INFO.num_cores
NS = _INFO.num_subcores
NW = NC * NS
N_ET = EMBED // 8
N_JT = SEQ // 128
NTILES = 31
STAGE = 28

_mesh = plsc.VectorSubcoreMesh(core_axis_name="c", subcore_axis_name="s")


@functools.partial(
    pl.kernel,
    out_type=jax.ShapeDtypeStruct((SEQ, N_ET, N_JT, 8, 128), jnp.float32),
    mesh=_mesh,
    compiler_params=pltpu.CompilerParams(needs_layout_passes=False),
    scratch_types=[
        pltpu.VMEM((NTILES, EMBED, 128), jnp.float32),
        pltpu.SemaphoreType.DMA,
        pltpu.SemaphoreType.DMA,
    ],
)
def _rpe_sc(table_hbm, out_hbm, btt_v, lsem, csem):
    wid = lax.axis_index("s") * NC + lax.axis_index("c")
    c0 = 4 * wid
    base0 = 127 - c0
    lane = lax.iota(jnp.int32, 16)

    pltpu.make_async_copy(table_hbm, btt_v.at[pl.ds(STAGE, 3)], lsem).start()
    pltpu.make_async_copy(table_hbm, btt_v.at[pl.ds(STAGE, 3)], lsem).wait()

    def staged_gather(widx):
        p = STAGE + lax.shift_right_logical(widx, 12)
        o = lax.bitwise_and(widx, 4095)
        return plsc.load_gather(
            btt_v,
            [p, lax.shift_right_logical(o, 7), lax.bitwise_and(o, 127)],
        )

    def _band_row(e, _):
        for t in range(3):
            for kk in range(8):
                m = base0 + 128 * (14 + t) + kk * 16 + lane
                ridx = jnp.clip(m, LO, HI) - LO
                btt_v[14 + t, e, pl.ds(kk * 16, 16)] = staged_gather(
                    ridx * EMBED + e)
        return 0
    lax.fori_loop(0, EMBED, _band_row, 0)

    def _const_row(e, _):
        vlo = staged_gather(jnp.full((16,), 0, jnp.int32) + e)
        vhi = staged_gather(jnp.full((16,), (VOCAB - 1) * EMBED, jnp.int32) + e)
        for p in range(14):
            for kk in range(8):
                btt_v[p, e, pl.ds(kk * 16, 16)] = vlo
        for p in range(17, STAGE):
            for kk in range(8):
                btt_v[p, e, pl.ds(kk * 16, 16)] = vhi
        return 0
    lax.fori_loop(0, EMBED, _const_row, 0)

    def _stage_row(e, _):
        vhi = btt_v[17, e, pl.ds(0, 16)]
        for p in range(STAGE, NTILES):
            for kk in range(8):
                btt_v[p, e, pl.ds(kk * 16, 16)] = vhi
        return 0
    lax.fori_loop(0, EMBED, _stage_row, 0)

    def copy_desc(d, c):
        k = lax.shift_right_logical(d, 2)
        et = lax.bitwise_and(d, 3)
        return pltpu.make_async_copy(
            btt_v.at[pl.ds(15 - k, 16), pl.ds(et * 8, 8), :],
            out_hbm.at[c + 128 * k, et],
            csem,
        )

    for p in range(4):
        c = c0 + p
        if p > 0:
            def _shift_row(e, _):
                ev = jnp.full((16,), 0, jnp.int32) + e
                for t in (2, 1, 0):
                    for kk in (7, 6, 5, 4, 3, 2, 1, 0):
                        g = 128 * t + kk * 16 + lane - 1
                        val = plsc.load_gather(
                            btt_v,
                            [14 + lax.shift_right_arithmetic(g, 7),
                             ev,
                             lax.bitwise_and(g, 127)],
                        )
                        btt_v[14 + t, e, pl.ds(kk * 16, 16)] = val
                return 0
            lax.fori_loop(0, EMBED, _shift_row, 0)

        def _start(d, _, c=c):
            copy_desc(d, c).start()
            return 0

        def _roll(d, _, c=c):
            copy_desc(d, c).start()
            copy_desc(d - 8, c).wait()
            return 0

        def _drain(d, _, c=c):
            copy_desc(d, c).wait()
            return 0

        lax.fori_loop(0, 8, _start, 0)
        lax.fori_loop(8, 64, _roll, 0)
        lax.fori_loop(56, 64, _drain, 0)


def kernel(table, seq_len):
    flat = jnp.pad(table.reshape(VOCAB * EMBED), (0, 3 * 4096 - VOCAB * EMBED))
    out5 = _rpe_sc(flat.reshape(3, EMBED, 128))
    return out5.transpose(0, 2, 4, 1, 3).reshape(SEQ, SEQ, EMBED)


# speedup vs baseline: 79.4766x; 1.0058x over previous
import functools

import jax
import jax.numpy as jnp
from jax import lax
from jax.experimental import pallas as pl
from jax.experimental.pallas import tpu as pltpu
from jax.experimental.pallas import tpu_sc as plsc

EMBED = 32
MAX_REL = 128
VOCAB = 2 * MAX_REL + 1
SEQ = 2048
LO = SEQ - 1 - MAX_REL
HI = SEQ - 1 + MAX_REL

_INFO = plsc.get_sparse_core_info()
NC = _INFO.num_cores
NS = _INFO.num_subcores
NW = NC * NS
N_ET = EMBED // 8
N_JT = SEQ // 128
NTILES = 31
STAGE = 28

_mesh = plsc.VectorSubcoreMesh(core_axis_name="c", subcore_axis_name="s")


@functools.partial(
    pl.kernel,
    out_type=jax.ShapeDtypeStruct((SEQ, N_ET, N_JT, 8, 128), jnp.float32),
    mesh=_mesh,
    compiler_params=pltpu.CompilerParams(needs_layout_passes=False),
    scratch_types=[
        pltpu.VMEM((NTILES, EMBED, 128), jnp.float32),
        pltpu.SemaphoreType.DMA,
        pltpu.SemaphoreType.DMA,
    ],
)
def _rpe_sc(table_hbm, out_hbm, btt_v, lsem, csem):
    wid = lax.axis_index("s") * NC + lax.axis_index("c")
    c0 = 4 * wid
    base0 = 127 - c0
    lane = lax.iota(jnp.int32, 16)

    pltpu.make_async_copy(table_hbm, btt_v.at[pl.ds(STAGE, 3)], lsem).start()
    pltpu.make_async_copy(table_hbm, btt_v.at[pl.ds(STAGE, 3)], lsem).wait()

    def staged_gather(widx):
        p = STAGE + lax.shift_right_logical(widx, 12)
        o = lax.bitwise_and(widx, 4095)
        return plsc.load_gather(
            btt_v,
            [p, lax.shift_right_logical(o, 7), lax.bitwise_and(o, 127)],
        )

    def _band_row(e, _):
        for t in range(3):
            for kk in range(8):
                m = base0 + 128 * (14 + t) + kk * 16 + lane
                ridx = jnp.clip(m, LO, HI) - LO
                btt_v[14 + t, e, pl.ds(kk * 16, 16)] = staged_gather(
                    ridx * EMBED + e)
        return 0
    lax.fori_loop(0, EMBED, _band_row, 0)

    def _hi_row(e, _):
        vhi = staged_gather(jnp.full((16,), (VOCAB - 1) * EMBED, jnp.int32) + e)
        for p in range(17, STAGE):
            for kk in range(8):
                btt_v[p, e, pl.ds(kk * 16, 16)] = vhi
        return 0
    lax.fori_loop(0, EMBED, _hi_row, 0)

    def _lo13_row(e, _):
        vlo = staged_gather(jnp.full((16,), 0, jnp.int32) + e)
        for kk in range(8):
            btt_v[13, e, pl.ds(kk * 16, 16)] = vlo
        return 0
    lax.fori_loop(0, EMBED, _lo13_row, 0)

    def _stage_row(e, _):
        vhi = btt_v[17, e, pl.ds(0, 16)]
        for p in range(STAGE, NTILES):
            for kk in range(8):
                btt_v[p, e, pl.ds(kk * 16, 16)] = vhi
        return 0
    lax.fori_loop(0, EMBED, _stage_row, 0)

    def _lo_fill():
        def _lo_row(e, _):
            vlo = btt_v[13, e, pl.ds(0, 16)]
            for p in range(13):
                for kk in range(8):
                    btt_v[p, e, pl.ds(kk * 16, 16)] = vlo
            return 0
        lax.fori_loop(0, EMBED, _lo_row, 0)

    def copy_desc(d, c):
        k = lax.shift_right_logical(d, 2)
        et = lax.bitwise_and(d, 3)
        return pltpu.make_async_copy(
            btt_v.at[pl.ds(15 - k, 16), pl.ds(et * 8, 8), :],
            out_hbm.at[c + 128 * k, et],
            csem,
        )

    for p in range(4):
        c = c0 + p
        if p > 0:
            def _shift_row(e, _):
                ev = jnp.full((16,), 0, jnp.int32) + e
                for t in (2, 1, 0):
                    for kk in (7, 6, 5, 4, 3, 2, 1, 0):
                        g = 128 * t + kk * 16 + lane - 1
                        val = plsc.load_gather(
                            btt_v,
                            [14 + lax.shift_right_arithmetic(g, 7),
                             ev,
                             lax.bitwise_and(g, 127)],
                        )
                        btt_v[14 + t, e, pl.ds(kk * 16, 16)] = val
                return 0
            lax.fori_loop(0, EMBED, _shift_row, 0)

        def _start(d, _, c=c):
            copy_desc(d, c).start()
            return 0

        def _roll(d, _, c=c):
            copy_desc(d, c).start()
            copy_desc(d - 8, c).wait()
            return 0

        def _drain(d, _, c=c):
            copy_desc(d, c).wait()
            return 0

        lax.fori_loop(0, 8, _start, 0)
        if p == 0:
            _lo_fill()
        lax.fori_loop(8, 64, _roll, 0)
        lax.fori_loop(56, 64, _drain, 0)


def kernel(table, seq_len):
    flat = jnp.pad(table.reshape(VOCAB * EMBED), (0, 3 * 4096 - VOCAB * EMBED))
    out5 = _rpe_sc(flat.reshape(3, EMBED, 128))
    return out5.transpose(0, 2, 4, 1, 3).reshape(SEQ, SEQ, EMBED)


# speedup vs baseline: 79.5644x; 1.0011x over previous
import functools

import jax
import jax.numpy as jnp
from jax import lax
from jax.experimental import pallas as pl
from jax.experimental.pallas import tpu as pltpu
from jax.experimental.pallas import tpu_sc as plsc

EMBED = 32
MAX_REL = 128
VOCAB = 2 * MAX_REL + 1
SEQ = 2048
LO = SEQ - 1 - MAX_REL
HI = SEQ - 1 + MAX_REL

_INFO = plsc.get_sparse_core_info()
NC = _INFO.num_cores
NS = _INFO.num_subcores
NW = NC * NS
N_ET = EMBED // 8
N_JT = SEQ // 128
NTILES = 31
STAGE = 28

_mesh = plsc.VectorSubcoreMesh(core_axis_name="c", subcore_axis_name="s")


@functools.partial(
    pl.kernel,
    out_type=jax.ShapeDtypeStruct((SEQ, N_ET, N_JT, 8, 128), jnp.float32),
    mesh=_mesh,
    compiler_params=pltpu.CompilerParams(needs_layout_passes=False),
    scratch_types=[
        pltpu.VMEM((NTILES, N_ET, 8, 128), jnp.float32),
        pltpu.SemaphoreType.DMA,
        pltpu.SemaphoreType.DMA,
    ],
)
def _rpe_sc(table_hbm, out_hbm, btt_v, lsem, csem):
    wid = lax.axis_index("s") * NC + lax.axis_index("c")
    c0 = 4 * wid
    base0 = 127 - c0
    lane = lax.iota(jnp.int32, 16)

    pltpu.make_async_copy(table_hbm, btt_v.at[pl.ds(STAGE, 3)], lsem).start()
    pltpu.make_async_copy(table_hbm, btt_v.at[pl.ds(STAGE, 3)], lsem).wait()

    def staged_gather(widx):
        p = STAGE + lax.shift_right_logical(widx, 12)
        o = lax.bitwise_and(widx, 4095)
        return plsc.load_gather(
            btt_v,
            [p,
             lax.shift_right_logical(o, 10),
             lax.bitwise_and(lax.shift_right_logical(o, 7), 7),
             lax.bitwise_and(o, 127)],
        )

    def band_gather(e):
        return lax.shift_right_logical(e, 3), lax.bitwise_and(e, 7)

    def _band_row(e, _):
        et, es = band_gather(e)
        for t in range(3):
            for kk in range(8):
                m = base0 + 128 * (14 + t) + kk * 16 + lane
                ridx = jnp.clip(m, LO, HI) - LO
                btt_v[14 + t, et, es, pl.ds(kk * 16, 16)] = staged_gather(
                    ridx * EMBED + e)
        return 0
    lax.fori_loop(0, EMBED, _band_row, 0)

    def _hi_row(e, _):
        et, es = band_gather(e)
        vhi = staged_gather(jnp.full((16,), (VOCAB - 1) * EMBED, jnp.int32) + e)
        for p in range(17, STAGE):
            for kk in range(8):
                btt_v[p, et, es, pl.ds(kk * 16, 16)] = vhi
        return 0
    lax.fori_loop(0, EMBED, _hi_row, 0)

    def _lo13_row(e, _):
        et, es = band_gather(e)
        vlo = staged_gather(jnp.full((16,), 0, jnp.int32) + e)
        for kk in range(8):
            btt_v[13, et, es, pl.ds(kk * 16, 16)] = vlo
        return 0
    lax.fori_loop(0, EMBED, _lo13_row, 0)

    def _stage_row(e, _):
        et, es = band_gather(e)
        vhi = btt_v[17, et, es, pl.ds(0, 16)]
        for p in range(STAGE, NTILES):
            for kk in range(8):
                btt_v[p, et, es, pl.ds(kk * 16, 16)] = vhi
        return 0
    lax.fori_loop(0, EMBED, _stage_row, 0)

    def _deferred_fills():
        def _lo_row(e, _):
            et, es = band_gather(e)
            vlo = btt_v[13, et, es, pl.ds(0, 16)]
            for p in range(13):
                for kk in range(8):
                    btt_v[p, et, es, pl.ds(kk * 16, 16)] = vlo
            return 0
        lax.fori_loop(0, EMBED, _lo_row, 0)

    def copy_desc(d, c):
        k = lax.shift_right_logical(d, 2)
        et = lax.bitwise_and(d, 3)
        return pltpu.make_async_copy(
            btt_v.at[pl.ds(15 - k, 16), et],
            out_hbm.at[c + 128 * k, et],
            csem,
        )

    for p in range(4):
        c = c0 + p
        if p > 0:
            def _shift_row(e, _):
                et, es = band_gather(e)
                etv = jnp.full((16,), 0, jnp.int32) + et
                esv = jnp.full((16,), 0, jnp.int32) + es
                for t in (2, 1, 0):
                    for kk in (7, 6, 5, 4, 3, 2, 1, 0):
                        g = 128 * t + kk * 16 + lane - 1
                        val = plsc.load_gather(
                            btt_v,
                            [14 + lax.shift_right_arithmetic(g, 7),
                             etv, esv,
                             lax.bitwise_and(g, 127)],
                        )
                        btt_v[14 + t, et, es, pl.ds(kk * 16, 16)] = val
                return 0
            lax.fori_loop(0, EMBED, _shift_row, 0)

        def _start(d, _, c=c):
            copy_desc(d, c).start()
            return 0

        def _roll(d, _, c=c):
            copy_desc(d, c).start()
            copy_desc(d - 8, c).wait()
            return 0

        def _drain(d, _, c=c):
            copy_desc(d, c).wait()
            return 0

        lax.fori_loop(0, 8, _start, 0)
        if p == 0:
            _deferred_fills()
        lax.fori_loop(8, 64, _roll, 0)
        lax.fori_loop(56, 64, _drain, 0)


def kernel(table, seq_len):
    flat = jnp.pad(table.reshape(VOCAB * EMBED), (0, 3 * 4096 - VOCAB * EMBED))
    out5 = _rpe_sc(flat.reshape(3, N_ET, 8, 128))
    return out5.transpose(0, 2, 4, 1, 3).reshape(SEQ, SEQ, EMBED)
